# baseline (speedup 1.0000x reference)
import functools

import jax
import jax.numpy as jnp
from jax import lax
from jax.experimental import pallas as pl
from jax.experimental.pallas import tpu as pltpu
from jax.experimental.pallas import tpu_sc as plsc

_N = 10000
_D = 128
_E = 320000
_KAPPA = 0.95

_NC = 2
_NS = 16
_CH = 128
_EPAD = 327680
_NCHUNK = _EPAD // _CH
_CPT_MAIN = _NCHUNK // _NS
_CPT_DEG = _NCHUNK // (_NC * _NS)
_NACC = 10240
_RPT = _NACC // _NS
_DW = 16
_BR = 1000
_DEG_DEPTH = 8


def _sc_mesh():
    return plsc.VectorSubcoreMesh(core_axis_name="c", subcore_axis_name="s")


def _sc_deg_body(dst_hbm, ones_hbm, zeros_hbm, deg_hbm, idx_v, ones_v, acc_sh, sem):
    c = lax.axis_index("c")
    s = lax.axis_index("s")
    pltpu.sync_copy(zeros_hbm, acc_sh.at[pl.ds(s * _RPT, _RPT)])
    pltpu.sync_copy(ones_hbm, ones_v)
    base = (c * _NS + s) * (_CPT_DEG * _CH)
    pltpu.sync_copy(dst_hbm.at[pl.ds(base, _CPT_DEG * _CH)], idx_v)
    plsc.subcore_barrier()

    def step(g, carry):
        idx = idx_v.at[pl.ds(g * _CH, _CH)]
        pltpu.async_copy(ones_v, acc_sh.at[idx], sem, add=True)

        @pl.when(g >= _DEG_DEPTH)
        def _():
            pltpu.make_async_copy(ones_v, acc_sh.at[idx], sem).wait()

        return carry

    lax.fori_loop(0, _CPT_DEG, step, 0)

    def drain(g, carry):
        pltpu.make_async_copy(ones_v, acc_sh.at[idx_v.at[pl.ds(0, _CH)]], sem).wait()
        return carry

    lax.fori_loop(0, _DEG_DEPTH, drain, 0)
    plsc.subcore_barrier()
    pltpu.sync_copy(acc_sh.at[pl.ds(s * _RPT, _RPT)],
                    deg_hbm.at[pl.ds(c * _NACC + s * _RPT, _RPT)])


def _sc_deg(dst2d, ones_m, zeros_m):
    return pl.kernel(
        _sc_deg_body,
        out_type=jax.ShapeDtypeStruct((_NC * _NACC, _D), jnp.float32),
        mesh=_sc_mesh(),
        scratch_types=[
            pltpu.VMEM((_CPT_DEG * _CH,), jnp.int32),
            pltpu.VMEM((_CH, _D), jnp.float32),
            pltpu.VMEM_SHARED((_NACC, _D), jnp.float32),
            pltpu.SemaphoreType.DMA,
        ],
        name="sc_deg",
    )(dst2d, ones_m, zeros_m)


def _sc_main_body(pk_hbm, t_hbm, zeros_hbm, out_hbm,
                  p0, p1, p2, p3, r0, r1,
                  sg0, sg1, ss0, ss1, sp0, sp1, sp2, sp3, acc_sh):
    c = lax.axis_index("c")
    s = lax.axis_index("s")
    rows = (r0, r1)
    sem_g = (sg0, sg1)
    sem_s = (ss0, ss1)
    pbuf = (p0, p1, p2, p3)
    sem_p = (sp0, sp1, sp2, sp3)
    pltpu.sync_copy(zeros_hbm, acc_sh.at[pl.ds(s * _RPT, _RPT)])
    cbase = (c * _NCHUNK + s * _CPT_MAIN) * (2 * _CH)
    pltpu.sync_copy(pk_hbm.at[pl.ds(cbase, 2 * _CH)], p0)
    plsc.subcore_barrier()

    last = _CPT_MAIN - 1
    pltpu.async_copy(t_hbm.at[p0.at[pl.ds(0, _CH)]], r0, sg0)
    pltpu.async_copy(pk_hbm.at[pl.ds(cbase + 2 * _CH, 2 * _CH)], p1, sp1)

    def outer(g, carry):
        for b in range(4):
            i = 4 * g + b
            b2 = b & 1
            nb2 = 1 - b2
            gidx = pbuf[b].at[pl.ds(0, _CH)]
            didx = pbuf[b].at[pl.ds(_CH, _CH)]
            pltpu.make_async_copy(t_hbm.at[gidx], rows[b2], sem_g[b2]).wait()
            pltpu.async_copy(rows[b2], acc_sh.at[didx], sem_s[b2], add=True)

            @pl.when(i < last)
            def _():
                @pl.when(i > 0)
                def _():
                    pltpu.make_async_copy(
                        rows[nb2], acc_sh.at[didx], sem_s[nb2]).wait()

                nxt = (b + 1) & 3
                pltpu.make_async_copy(
                    pk_hbm.at[pl.ds(cbase, 2 * _CH)], pbuf[nxt], sem_p[nxt]).wait()
                pltpu.async_copy(
                    t_hbm.at[pbuf[nxt].at[pl.ds(0, _CH)]], rows[nb2], sem_g[nb2])

                @pl.when(i + 2 <= last)
                def _():
                    n2 = (b + 2) & 3
                    pltpu.async_copy(
                        pk_hbm.at[pl.ds(cbase + (i + 2) * 2 * _CH, 2 * _CH)],
                        pbuf[n2], sem_p[n2])

        return carry

    lax.fori_loop(0, _CPT_MAIN // 4, outer, 0)
    dsl0 = pbuf[0].at[pl.ds(_CH, _CH)]
    pltpu.make_async_copy(r0, acc_sh.at[dsl0], ss0).wait()
    pltpu.make_async_copy(r1, acc_sh.at[dsl0], ss1).wait()
    plsc.subcore_barrier()
    pltpu.sync_copy(acc_sh.at[pl.ds(s * _RPT, _RPT)],
                    out_hbm.at[pl.ds(c * _NACC + s * _RPT, _RPT)])


def _sc_main(packed, table, zeros_m):
    return pl.kernel(
        _sc_main_body,
        out_type=jax.ShapeDtypeStruct((_NC * _NACC, _D), jnp.float32),
        mesh=_sc_mesh(),
        scratch_types=[
            pltpu.VMEM((2 * _CH,), jnp.int32),
            pltpu.VMEM((2 * _CH,), jnp.int32),
            pltpu.VMEM((2 * _CH,), jnp.int32),
            pltpu.VMEM((2 * _CH,), jnp.int32),
            pltpu.VMEM((_CH, _D), jnp.float32),
            pltpu.VMEM((_CH, _D), jnp.float32),
            pltpu.SemaphoreType.DMA,
            pltpu.SemaphoreType.DMA,
            pltpu.SemaphoreType.DMA,
            pltpu.SemaphoreType.DMA,
            pltpu.SemaphoreType.DMA,
            pltpu.SemaphoreType.DMA,
            pltpu.SemaphoreType.DMA,
            pltpu.SemaphoreType.DMA,
            pltpu.VMEM_SHARED((_NACC, _D), jnp.float32),
        ],
        name="sc_main",
    )(packed, table, zeros_m)


def _tc_prep1_body(feat_ref, emb_ref, wg_ref, fm_ref, h_ref, e_ref):
    F = fm_ref[...]
    W = lax.dot_general(F, F, (((0,), (0,)), ((), ())),
                        preferred_element_type=jnp.float32)
    W = W * (_KAPPA / (jnp.sqrt(jnp.sum(W * W)) + 1e-5))
    h_ref[...] = jnp.dot(feat_ref[...], wg_ref[...],
                         preferred_element_type=jnp.float32)
    e_ref[...] = jnp.dot(emb_ref[...], W, preferred_element_type=jnp.float32)


def _tc_prep1(features, embeddings, W_gcn, Fmat):
    grid = (_N // _BR,)
    return pl.pallas_call(
        _tc_prep1_body,
        grid=grid,
        in_specs=[
            pl.BlockSpec((_BR, _D), lambda i: (i, 0)),
            pl.BlockSpec((_BR, _D), lambda i: (i, 0)),
            pl.BlockSpec((_D, _D), lambda i: (0, 0)),
            pl.BlockSpec((_D, _D), lambda i: (0, 0)),
        ],
        out_specs=[
            pl.BlockSpec((_BR, _D), lambda i: (i, 0)),
            pl.BlockSpec((_BR, _D), lambda i: (i, 0)),
        ],
        out_shape=[
            jax.ShapeDtypeStruct((_N, _D), jnp.float32),
            jax.ShapeDtypeStruct((_N, _D), jnp.float32),
        ],
        name="tc_prep1",
    )(features, embeddings, W_gcn, Fmat)


def _tc_prep2_body(h_ref, e_ref, b_ref, deg_ref, t_ref, base_ref):
    dv16 = lax.rsqrt(deg_ref[0] + deg_ref[1] + 1.0)
    dv = dv16[:, 0:1]
    h = h_ref[...]
    t_ref[0] = dv * h
    t_ref[1] = e_ref[...]
    base_ref[...] = (dv * dv) * h + b_ref[...]


def _tc_prep2(h, embW, b_row, deg3):
    grid = (_N // _BR,)
    return pl.pallas_call(
        _tc_prep2_body,
        grid=grid,
        in_specs=[
            pl.BlockSpec((_BR, _D), lambda i: (i, 0)),
            pl.BlockSpec((_BR, _D), lambda i: (i, 0)),
            pl.BlockSpec((1, _D), lambda i: (0, 0)),
            pl.BlockSpec((_NC, _BR, _DW), lambda i: (0, i, 0)),
        ],
        out_specs=[
            pl.BlockSpec((_NC, _BR, _D), lambda i: (0, i, 0)),
            pl.BlockSpec((_BR, _D), lambda i: (i, 0)),
        ],
        out_shape=[
            jax.ShapeDtypeStruct((_NC, _N, _D), jnp.float32),
            jax.ShapeDtypeStruct((_N, _D), jnp.float32),
        ],
        name="tc_prep2",
    )(h, embW, b_row, deg3)


def _tc_combine_body(acc_ref, base_ref, deg_ref, out_ref):
    dv16 = lax.rsqrt(deg_ref[0] + deg_ref[1] + 1.0)
    dv = dv16[:, 0:1]
    out_ref[...] = dv * acc_ref[0] + acc_ref[1] + base_ref[...]


def _tc_combine(acc3, base, deg3):
    grid = (_N // _BR,)
    return pl.pallas_call(
        _tc_combine_body,
        grid=grid,
        in_specs=[
            pl.BlockSpec((_NC, _BR, _D), lambda i: (0, i, 0)),
            pl.BlockSpec((_BR, _D), lambda i: (i, 0)),
            pl.BlockSpec((_NC, _BR, _DW), lambda i: (0, i, 0)),
        ],
        out_specs=pl.BlockSpec((_BR, _D), lambda i: (i, 0)),
        out_shape=jax.ShapeDtypeStruct((_N, _D), jnp.float32),
        name="tc_combine",
    )(acc3, base, deg3)


def kernel(features, sparse_adj, W_gcn, b_gcn, Fmat, embeddings):
    src = sparse_adj[0].astype(jnp.int32)
    dst = sparse_adj[1].astype(jnp.int32)
    npad = _EPAD - _E
    pad_i = jnp.arange(npad, dtype=jnp.int32)
    src_p = jnp.concatenate([src, pad_i % _N])
    dst_p = jnp.concatenate([dst, _N + pad_i % (_NACC - _N)])
    srcoff3 = jnp.stack([src_p, src_p + _N]).reshape(_NC, _NCHUNK, _CH)
    dst3 = jnp.broadcast_to(dst_p.reshape(1, _NCHUNK, _CH), (_NC, _NCHUNK, _CH))
    packed = jnp.concatenate([srcoff3, dst3], axis=2).reshape(-1)
    zeros_m = jnp.zeros((_RPT, _D), jnp.float32)
    ones_m = jnp.ones((_CH, _D), jnp.float32)

    deg_flat = _sc_deg(dst_p, ones_m, zeros_m)
    h, embW = _tc_prep1(features, embeddings, W_gcn, Fmat)
    deg3 = deg_flat.reshape(_NC, _NACC, _D)[:, :, :_DW]
    t3, base = _tc_prep2(h, embW, b_gcn.reshape(1, _D), deg3)
    table = t3.reshape(_NC * _N, _D)
    acc_flat = _sc_main(packed, table, zeros_m)
    acc3 = acc_flat.reshape(_NC, _NACC, _D)
    return _tc_combine(acc3, base, deg3)


# speedup vs baseline: 1.0005x; 1.0005x over previous
import functools

import jax
import jax.numpy as jnp
from jax import lax
from jax.experimental import pallas as pl
from jax.experimental.pallas import tpu as pltpu
from jax.experimental.pallas import tpu_sc as plsc

_N = 10000
_D = 128
_E = 320000
_KAPPA = 0.95

_NC = 2
_NS = 16
_CH = 128
_EPAD = 327680
_NCHUNK = _EPAD // _CH
_CPT_MAIN = _NCHUNK // _NS
_CPT_DEG = _NCHUNK // (_NC * _NS)
_NACC = 10240
_RPT = _NACC // _NS
_DW = 16
_BR = 1000
_DEG_DEPTH = 16


def _sc_mesh():
    return plsc.VectorSubcoreMesh(core_axis_name="c", subcore_axis_name="s")


def _sc_deg_body(dst_hbm, ones_hbm, zeros_hbm, deg_hbm, idx_v, ones_v, acc_sh, sem):
    c = lax.axis_index("c")
    s = lax.axis_index("s")
    pltpu.sync_copy(zeros_hbm, acc_sh.at[pl.ds(s * _RPT, _RPT)])
    pltpu.sync_copy(ones_hbm, ones_v)
    base = (c * _NS + s) * (_CPT_DEG * _CH)
    pltpu.sync_copy(dst_hbm.at[pl.ds(base, _CPT_DEG * _CH)], idx_v)
    plsc.subcore_barrier()

    def step(g, carry):
        idx = idx_v.at[pl.ds(g * _CH, _CH)]
        pltpu.async_copy(ones_v, acc_sh.at[idx], sem, add=True)

        @pl.when(g >= _DEG_DEPTH)
        def _():
            pltpu.make_async_copy(ones_v, acc_sh.at[idx], sem).wait()

        return carry

    lax.fori_loop(0, _CPT_DEG, step, 0)

    def drain(g, carry):
        pltpu.make_async_copy(ones_v, acc_sh.at[idx_v.at[pl.ds(0, _CH)]], sem).wait()
        return carry

    lax.fori_loop(0, _DEG_DEPTH, drain, 0)
    plsc.subcore_barrier()
    pltpu.sync_copy(acc_sh.at[pl.ds(s * _RPT, _RPT)],
                    deg_hbm.at[pl.ds(c * _NACC + s * _RPT, _RPT)])


def _sc_deg(dst2d, ones_m, zeros_m):
    return pl.kernel(
        _sc_deg_body,
        out_type=jax.ShapeDtypeStruct((_NC * _NACC, _D), jnp.float32),
        mesh=_sc_mesh(),
        scratch_types=[
            pltpu.VMEM((_CPT_DEG * _CH,), jnp.int32),
            pltpu.VMEM((_CH, _D), jnp.float32),
            pltpu.VMEM_SHARED((_NACC, _D), jnp.float32),
            pltpu.SemaphoreType.DMA,
        ],
        name="sc_deg",
    )(dst2d, ones_m, zeros_m)


def _sc_main_body(pk_hbm, t_hbm, zeros_hbm, out_hbm,
                  p0, p1, p2, p3, r0, r1,
                  sg0, sg1, ss0, ss1, sp0, sp1, sp2, sp3, acc_sh):
    c = lax.axis_index("c")
    s = lax.axis_index("s")
    rows = (r0, r1)
    sem_g = (sg0, sg1)
    sem_s = (ss0, ss1)
    pbuf = (p0, p1, p2, p3)
    sem_p = (sp0, sp1, sp2, sp3)
    pltpu.sync_copy(zeros_hbm, acc_sh.at[pl.ds(s * _RPT, _RPT)])
    cbase = (c * _NCHUNK + s * _CPT_MAIN) * (2 * _CH)
    pltpu.sync_copy(pk_hbm.at[pl.ds(cbase, 2 * _CH)], p0)
    plsc.subcore_barrier()

    last = _CPT_MAIN - 1
    pltpu.async_copy(t_hbm.at[p0.at[pl.ds(0, _CH)]], r0, sg0)
    pltpu.async_copy(pk_hbm.at[pl.ds(cbase + 2 * _CH, 2 * _CH)], p1, sp1)

    def outer(g, carry):
        for b in range(4):
            i = 4 * g + b
            b2 = b & 1
            nb2 = 1 - b2
            gidx = pbuf[b].at[pl.ds(0, _CH)]
            didx = pbuf[b].at[pl.ds(_CH, _CH)]
            pltpu.make_async_copy(t_hbm.at[gidx], rows[b2], sem_g[b2]).wait()
            pltpu.async_copy(rows[b2], acc_sh.at[didx], sem_s[b2], add=True)

            @pl.when(i < last)
            def _():
                @pl.when(i > 0)
                def _():
                    pltpu.make_async_copy(
                        rows[nb2], acc_sh.at[didx], sem_s[nb2]).wait()

                nxt = (b + 1) & 3
                pltpu.make_async_copy(
                    pk_hbm.at[pl.ds(cbase, 2 * _CH)], pbuf[nxt], sem_p[nxt]).wait()
                pltpu.async_copy(
                    t_hbm.at[pbuf[nxt].at[pl.ds(0, _CH)]], rows[nb2], sem_g[nb2])

                @pl.when(i + 2 <= last)
                def _():
                    n2 = (b + 2) & 3
                    pltpu.async_copy(
                        pk_hbm.at[pl.ds(cbase + (i + 2) * 2 * _CH, 2 * _CH)],
                        pbuf[n2], sem_p[n2])

        return carry

    lax.fori_loop(0, _CPT_MAIN // 4, outer, 0)
    dsl0 = pbuf[0].at[pl.ds(_CH, _CH)]
    pltpu.make_async_copy(r0, acc_sh.at[dsl0], ss0).wait()
    pltpu.make_async_copy(r1, acc_sh.at[dsl0], ss1).wait()
    plsc.subcore_barrier()
    pltpu.sync_copy(acc_sh.at[pl.ds(s * _RPT, _RPT)],
                    out_hbm.at[pl.ds(c * _NACC + s * _RPT, _RPT)])


def _sc_main(packed, table, zeros_m):
    return pl.kernel(
        _sc_main_body,
        out_type=jax.ShapeDtypeStruct((_NC * _NACC, _D), jnp.float32),
        mesh=_sc_mesh(),
        scratch_types=[
            pltpu.VMEM((2 * _CH,), jnp.int32),
            pltpu.VMEM((2 * _CH,), jnp.int32),
            pltpu.VMEM((2 * _CH,), jnp.int32),
            pltpu.VMEM((2 * _CH,), jnp.int32),
            pltpu.VMEM((_CH, _D), jnp.float32),
            pltpu.VMEM((_CH, _D), jnp.float32),
            pltpu.SemaphoreType.DMA,
            pltpu.SemaphoreType.DMA,
            pltpu.SemaphoreType.DMA,
            pltpu.SemaphoreType.DMA,
            pltpu.SemaphoreType.DMA,
            pltpu.SemaphoreType.DMA,
            pltpu.SemaphoreType.DMA,
            pltpu.SemaphoreType.DMA,
            pltpu.VMEM_SHARED((_NACC, _D), jnp.float32),
        ],
        name="sc_main",
    )(packed, table, zeros_m)


def _tc_prep1_body(feat_ref, emb_ref, wg_ref, fm_ref, h_ref, e_ref):
    F = fm_ref[...]
    W = lax.dot_general(F, F, (((0,), (0,)), ((), ())),
                        preferred_element_type=jnp.float32)
    W = W * (_KAPPA / (jnp.sqrt(jnp.sum(W * W)) + 1e-5))
    h_ref[...] = jnp.dot(feat_ref[...], wg_ref[...],
                         preferred_element_type=jnp.float32)
    e_ref[...] = jnp.dot(emb_ref[...], W, preferred_element_type=jnp.float32)


def _tc_prep1(features, embeddings, W_gcn, Fmat):
    grid = (_N // _BR,)
    return pl.pallas_call(
        _tc_prep1_body,
        grid=grid,
        in_specs=[
            pl.BlockSpec((_BR, _D), lambda i: (i, 0)),
            pl.BlockSpec((_BR, _D), lambda i: (i, 0)),
            pl.BlockSpec((_D, _D), lambda i: (0, 0)),
            pl.BlockSpec((_D, _D), lambda i: (0, 0)),
        ],
        out_specs=[
            pl.BlockSpec((_BR, _D), lambda i: (i, 0)),
            pl.BlockSpec((_BR, _D), lambda i: (i, 0)),
        ],
        out_shape=[
            jax.ShapeDtypeStruct((_N, _D), jnp.float32),
            jax.ShapeDtypeStruct((_N, _D), jnp.float32),
        ],
        name="tc_prep1",
    )(features, embeddings, W_gcn, Fmat)


def _tc_prep2_body(h_ref, e_ref, b_ref, deg_ref, t_ref, base_ref):
    dv16 = lax.rsqrt(deg_ref[0] + deg_ref[1] + 1.0)
    dv = dv16[:, 0:1]
    h = h_ref[...]
    t_ref[0] = dv * h
    t_ref[1] = e_ref[...]
    base_ref[...] = (dv * dv) * h + b_ref[...]


def _tc_prep2(h, embW, b_row, deg3):
    grid = (_N // _BR,)
    return pl.pallas_call(
        _tc_prep2_body,
        grid=grid,
        in_specs=[
            pl.BlockSpec((_BR, _D), lambda i: (i, 0)),
            pl.BlockSpec((_BR, _D), lambda i: (i, 0)),
            pl.BlockSpec((1, _D), lambda i: (0, 0)),
            pl.BlockSpec((_NC, _BR, _DW), lambda i: (0, i, 0)),
        ],
        out_specs=[
            pl.BlockSpec((_NC, _BR, _D), lambda i: (0, i, 0)),
            pl.BlockSpec((_BR, _D), lambda i: (i, 0)),
        ],
        out_shape=[
            jax.ShapeDtypeStruct((_NC, _N, _D), jnp.float32),
            jax.ShapeDtypeStruct((_N, _D), jnp.float32),
        ],
        name="tc_prep2",
    )(h, embW, b_row, deg3)


def _tc_combine_body(acc_ref, base_ref, deg_ref, out_ref):
    dv16 = lax.rsqrt(deg_ref[0] + deg_ref[1] + 1.0)
    dv = dv16[:, 0:1]
    out_ref[...] = dv * acc_ref[0] + acc_ref[1] + base_ref[...]


def _tc_combine(acc3, base, deg3):
    grid = (_N // _BR,)
    return pl.pallas_call(
        _tc_combine_body,
        grid=grid,
        in_specs=[
            pl.BlockSpec((_NC, _BR, _D), lambda i: (0, i, 0)),
            pl.BlockSpec((_BR, _D), lambda i: (i, 0)),
            pl.BlockSpec((_NC, _BR, _DW), lambda i: (0, i, 0)),
        ],
        out_specs=pl.BlockSpec((_BR, _D), lambda i: (i, 0)),
        out_shape=jax.ShapeDtypeStruct((_N, _D), jnp.float32),
        name="tc_combine",
    )(acc3, base, deg3)


def kernel(features, sparse_adj, W_gcn, b_gcn, Fmat, embeddings):
    src = sparse_adj[0].astype(jnp.int32)
    dst = sparse_adj[1].astype(jnp.int32)
    npad = _EPAD - _E
    pad_i = jnp.arange(npad, dtype=jnp.int32)
    src_p = jnp.concatenate([src, pad_i % _N])
    dst_p = jnp.concatenate([dst, _N + pad_i % (_NACC - _N)])
    srcoff3 = jnp.stack([src_p, src_p + _N]).reshape(_NC, _NCHUNK, _CH)
    dst3 = jnp.broadcast_to(dst_p.reshape(1, _NCHUNK, _CH), (_NC, _NCHUNK, _CH))
    packed = jnp.concatenate([srcoff3, dst3], axis=2).reshape(-1)
    zeros_m = jnp.zeros((_RPT, _D), jnp.float32)
    ones_m = jnp.ones((_CH, _D), jnp.float32)

    deg_flat = _sc_deg(dst_p, ones_m, zeros_m)
    h, embW = _tc_prep1(features, embeddings, W_gcn, Fmat)
    deg3 = deg_flat.reshape(_NC, _NACC, _D)[:, :, :_DW]
    t3, base = _tc_prep2(h, embW, b_gcn.reshape(1, _D), deg3)
    table = t3.reshape(_NC * _N, _D)
    acc_flat = _sc_main(packed, table, zeros_m)
    acc3 = acc_flat.reshape(_NC, _NACC, _D)
    return _tc_combine(acc3, base, deg3)
